# Initial kernel scaffold; baseline (speedup 1.0000x reference)
import jax
import jax.numpy as jnp
from jax import lax
from jax.experimental import pallas as pl
from jax.experimental.pallas import tpu as pltpu
from jax.experimental.pallas import tpu_sc as plsc

NU = 4000
NI = 6000
NN = NU + NI
NE = 320000
D = 128
NLAYER = 3

NC = 2
NS = 16
NW = NC * NS
CH = 128
EPT = 10240
EPAD = NW * EPT
NCHUNK = EPT // CH
RPT = NN // NS


def _sc_layer_body(x_hbm, src_hbm, dst_hbm, w_hbm, part_hbm,
                   src_v, dst_v, w_v, rows_v, acc, sem):
    cid = lax.axis_index("c")
    sid = lax.axis_index("s")

    def zero_row(r, carry):
        for j in range(8):
            rows_v[r, pl.ds(16 * j, 16)] = jnp.zeros((16,), jnp.float32)
        return carry

    lax.fori_loop(0, CH, zero_row, 0)

    abase = sid * RPT
    for k in range(4):
        pltpu.sync_copy(rows_v, acc.at[pl.ds(abase + k * CH, CH)])
    pltpu.sync_copy(rows_v.at[pl.ds(0, RPT - 4 * CH)],
                    acc.at[pl.ds(abase + 4 * CH, RPT - 4 * CH)])
    plsc.subcore_barrier()

    ebase = (cid * NS + sid) * EPT

    def chunk(c, carry):
        off = ebase + c * CH
        pltpu.sync_copy(src_hbm.at[pl.ds(off, CH)], src_v)
        pltpu.sync_copy(dst_hbm.at[pl.ds(off, CH)], dst_v)
        pltpu.sync_copy(w_hbm.at[pl.ds(off, CH)], w_v)
        pltpu.async_copy(x_hbm.at[src_v], rows_v, sem).wait()

        def scale(r, c2):
            wspl = plsc.load_gather(w_v, [jnp.full((16,), r, jnp.int32)])
            for j in range(8):
                rows_v[r, pl.ds(16 * j, 16)] = rows_v[r, pl.ds(16 * j, 16)] * wspl
            return c2

        lax.fori_loop(0, CH, scale, 0)
        pltpu.sync_copy(rows_v, acc.at[dst_v], add=True)
        return carry

    lax.fori_loop(0, NCHUNK, chunk, 0)
    plsc.subcore_barrier()

    pltpu.sync_copy(acc.at[pl.ds(abase, RPT)],
                    part_hbm.at[pl.ds(cid * NN + abase, RPT)])


@jax.jit
def _sc_layer(x, src, dst, w):
    mesh = plsc.VectorSubcoreMesh(core_axis_name="c", subcore_axis_name="s")
    return pl.kernel(
        _sc_layer_body,
        out_type=jax.ShapeDtypeStruct((NC * NN, D), jnp.float32),
        mesh=mesh,
        scratch_types=[
            pltpu.VMEM((CH,), jnp.int32),
            pltpu.VMEM((CH,), jnp.int32),
            pltpu.VMEM((CH,), jnp.float32),
            pltpu.VMEM((CH, D), jnp.float32),
            pltpu.VMEM_SHARED((NN, D), jnp.float32),
            pltpu.SemaphoreType.DMA,
        ],
    )(x, src, dst, w)


def _combine_body(p0_ref, p1_ref, a_ref, x_ref, ao_ref):
    s = p0_ref[...] + p1_ref[...]
    x_ref[...] = s
    ao_ref[...] = a_ref[...] + s


def _final_body(p0_ref, p1_ref, a_ref, m_ref):
    m_ref[...] = (a_ref[...] + p0_ref[...] + p1_ref[...]) * 0.25


_BLK = 1250


def _row_spec():
    return pl.BlockSpec((_BLK, D), lambda i: (i, 0))


@jax.jit
def _combine(p0, p1, a):
    return pl.pallas_call(
        _combine_body,
        grid=(NN // _BLK,),
        in_specs=[_row_spec(), _row_spec(), _row_spec()],
        out_specs=[_row_spec(), _row_spec()],
        out_shape=[jax.ShapeDtypeStruct((NN, D), jnp.float32)] * 2,
    )(p0, p1, a)


@jax.jit
def _finalize(p0, p1, a):
    return pl.pallas_call(
        _final_body,
        grid=(NN // _BLK,),
        in_specs=[_row_spec(), _row_spec(), _row_spec()],
        out_specs=_row_spec(),
        out_shape=jax.ShapeDtypeStruct((NN, D), jnp.float32),
    )(p0, p1, a)


def kernel(user_emb, item_emb, edge_weight, edge_index):
    x0 = jnp.concatenate([user_emb, item_emb], axis=0)
    pad = EPAD - NE
    src = jnp.pad(edge_index[1], (0, pad))
    dst = jnp.pad(edge_index[0], (0, pad))
    w = jnp.pad(edge_weight, (0, pad))

    x = x0
    acc = x0
    for layer in range(NLAYER):
        part = _sc_layer(x, src, dst, w)
        p0 = part[:NN]
        p1 = part[NN:]
        if layer < NLAYER - 1:
            x, acc = _combine(p0, p1, acc)
        else:
            mean = _finalize(p0, p1, acc)
    return (mean[:NU], mean[NU:])


# speedup vs baseline: 2.3386x; 2.3386x over previous
import jax
import jax.numpy as jnp
from jax import lax
from jax.experimental import pallas as pl
from jax.experimental.pallas import tpu as pltpu
from jax.experimental.pallas import tpu_sc as plsc

NU = 4000
NI = 6000
NN = NU + NI
NE = 320000
D = 128
NLAYER = 3

NC = 2
NS = 16
NW = NC * NS
CH = 128
EPT = 10240
EPAD = NW * EPT
NCHUNK = EPT // CH
NP = 10240
RPT = NP // NS


def _sc_layer_body(x_hbm, src_hbm, dst_hbm, w_hbm, part_hbm,
                   src_v, dst_v, w_v, rows_v, acc, sem):
    cid = lax.axis_index("c")
    sid = lax.axis_index("s")

    def zero_row(r, carry):
        for j in range(8):
            rows_v[r, pl.ds(16 * j, 16)] = jnp.zeros((16,), jnp.float32)
        return carry

    lax.fori_loop(0, CH, zero_row, 0)

    abase = sid * RPT
    for k in range(5):
        pltpu.sync_copy(rows_v, acc.at[pl.ds(abase + k * CH, CH)])
    plsc.subcore_barrier()

    ebase = (cid * NS + sid) * EPT

    def chunk(c, carry):
        off = ebase + c * CH
        pltpu.sync_copy(src_hbm.at[pl.ds(off, CH)], src_v)
        pltpu.sync_copy(dst_hbm.at[pl.ds(off, CH)], dst_v)
        pltpu.sync_copy(w_hbm.at[pl.ds(off, CH)], w_v)
        pltpu.async_copy(x_hbm.at[src_v], rows_v, sem).wait()

        def scale(g, c2):
            r0 = g * 16
            wvec = w_v[pl.ds(r0, 16)]
            for lane in range(16):
                wspl = jnp.full((16,), wvec[lane], jnp.float32)
                for j in range(8):
                    rows_v[r0 + lane, pl.ds(16 * j, 16)] = (
                        rows_v[r0 + lane, pl.ds(16 * j, 16)] * wspl)
            return c2

        lax.fori_loop(0, CH // 16, scale, 0)
        pltpu.sync_copy(rows_v, acc.at[dst_v], add=True)
        return carry

    lax.fori_loop(0, NCHUNK, chunk, 0)
    plsc.subcore_barrier()

    pltpu.sync_copy(acc.at[pl.ds(abase, RPT)],
                    part_hbm.at[pl.ds(cid * NP + abase, RPT)])


@jax.jit
def _sc_layer(x, src, dst, w):
    mesh = plsc.VectorSubcoreMesh(core_axis_name="c", subcore_axis_name="s")
    return pl.kernel(
        _sc_layer_body,
        out_type=jax.ShapeDtypeStruct((NC * NP, D), jnp.float32),
        mesh=mesh,
        scratch_types=[
            pltpu.VMEM((CH,), jnp.int32),
            pltpu.VMEM((CH,), jnp.int32),
            pltpu.VMEM((CH,), jnp.float32),
            pltpu.VMEM((CH, D), jnp.float32),
            pltpu.VMEM_SHARED((NP, D), jnp.float32),
            pltpu.SemaphoreType.DMA,
        ],
    )(x, src, dst, w)


def _combine_body(p0_ref, p1_ref, a_ref, x_ref, ao_ref):
    s = p0_ref[...] + p1_ref[...]
    x_ref[...] = s
    ao_ref[...] = a_ref[...] + s


def _final_body(p0_ref, p1_ref, a_ref, m_ref):
    m_ref[...] = (a_ref[...] + p0_ref[...] + p1_ref[...]) * 0.25


_BLK = 1280


def _row_spec():
    return pl.BlockSpec((_BLK, D), lambda i: (i, 0))


@jax.jit
def _combine(p0, p1, a):
    return pl.pallas_call(
        _combine_body,
        grid=(NP // _BLK,),
        in_specs=[_row_spec(), _row_spec(), _row_spec()],
        out_specs=[_row_spec(), _row_spec()],
        out_shape=[jax.ShapeDtypeStruct((NP, D), jnp.float32)] * 2,
    )(p0, p1, a)


@jax.jit
def _finalize(p0, p1, a):
    return pl.pallas_call(
        _final_body,
        grid=(NP // _BLK,),
        in_specs=[_row_spec(), _row_spec(), _row_spec()],
        out_specs=_row_spec(),
        out_shape=jax.ShapeDtypeStruct((NP, D), jnp.float32),
    )(p0, p1, a)


def kernel(user_emb, item_emb, edge_weight, edge_index):
    x0 = jnp.pad(jnp.concatenate([user_emb, item_emb], axis=0),
                 ((0, NP - NN), (0, 0)))
    pad = EPAD - NE
    src = jnp.pad(edge_index[1], (0, pad))
    dst = jnp.pad(edge_index[0], (0, pad))
    w = jnp.pad(edge_weight, (0, pad))

    x = x0
    acc = x0
    for layer in range(NLAYER):
        part = _sc_layer(x, src, dst, w)
        p0 = part[:NP]
        p1 = part[NP:]
        if layer < NLAYER - 1:
            x, acc = _combine(p0, p1, acc)
        else:
            mean = _finalize(p0, p1, acc)
    return (mean[:NU], mean[NU:NN])
